# baseline (speedup 1.0000x reference)
import functools

import jax
import jax.numpy as jnp
from jax import lax
from jax.experimental import pallas as pl
from jax.experimental.pallas import tpu as pltpu
from jax.experimental.pallas import tpu_sc as plsc

N = 10000
E = 320000
DF = 128
DE = 16
H = 64
HP = 128
NC = 2
NS = 16
NW = NC * NS
CE = 96
CPT = 106
E2 = NW * CPT * CE
NP = 10112
RPT = NP // NS

_sc_mesh = plsc.VectorSubcoreMesh(core_axis_name="c", subcore_axis_name="s")
_sc_params = pltpu.CompilerParams(needs_layout_passes=False)


@functools.partial(
    pl.kernel,
    out_type=jax.ShapeDtypeStruct((NC, NP, HP), jnp.float32),
    mesh=_sc_mesh,
    compiler_params=_sc_params,
    scratch_types=[
        pltpu.VMEM((N,), jnp.float32),
        pltpu.VMEM((N,), jnp.float32),
        pltpu.VMEM((CE,), jnp.int32),
        pltpu.VMEM((CE,), jnp.int32),
        pltpu.VMEM((CE,), jnp.float32),
        pltpu.VMEM((CE,), jnp.float32),
        pltpu.VMEM((CE, HP), jnp.float32),
        pltpu.VMEM((CE,), jnp.int32),
        pltpu.VMEM((CE,), jnp.int32),
        pltpu.VMEM((CE,), jnp.float32),
        pltpu.VMEM((CE,), jnp.float32),
        pltpu.VMEM((CE, HP), jnp.float32),
        pltpu.VMEM_SHARED((NP, HP), jnp.float32),
        pltpu.SemaphoreType.DMA,
        pltpu.SemaphoreType.DMA,
        pltpu.SemaphoreType.DMA,
        pltpu.SemaphoreType.DMA,
        pltpu.SemaphoreType.DMA,
        pltpu.SemaphoreType.DMA,
    ],
)
def _gat_edge_pass(src_hbm, dst_hbm, ae_hbm, asrc_hbm, adst_hbm, ht_hbm,
                   zro_hbm, out_hbm, asrc_v, adst_v,
                   src_a, dst_a, ae_a, ex_a, rows_a,
                   src_b, dst_b, ae_b, ex_b, rows_b,
                   acc_sh, sga, sgb, ssa, ssb, sla, slb):
    c = lax.axis_index("c")
    s = lax.axis_index("s")
    w = s * NC + c

    pltpu.sync_copy(zro_hbm, acc_sh.at[pl.ds(s * RPT, RPT)])

    pltpu.sync_copy(asrc_hbm, asrc_v)
    pltpu.sync_copy(adst_hbm, adst_v)
    plsc.subcore_barrier()

    def _base(j):
        return (w + j * NW) * CE

    def _load(j, sv, dv, av, sem):
        b = _base(j)
        pltpu.async_copy(src_hbm.at[pl.ds(b, CE)], sv, sem)
        pltpu.async_copy(dst_hbm.at[pl.ds(b, CE)], dv, sem)
        pltpu.async_copy(ae_hbm.at[pl.ds(b, CE)], av, sem)

    def _wait_load(j, sv, dv, av, sem):
        b = _base(j)
        pltpu.make_async_copy(src_hbm.at[pl.ds(b, CE)], sv, sem).wait()
        pltpu.make_async_copy(dst_hbm.at[pl.ds(b, CE)], dv, sem).wait()
        pltpu.make_async_copy(ae_hbm.at[pl.ds(b, CE)], av, sem).wait()

    def _ex_compute(sv, dv, av, xv):
        for g in range(CE // 16):
            sl = pl.ds(g * 16, 16)
            lg = (plsc.load_gather(asrc_v, [sv[sl]])
                  + plsc.load_gather(adst_v, [dv[sl]]) + av[sl])
            lg = jnp.where(lg > 0, lg, 0.2 * lg)
            xv[sl] = jnp.exp(lg)

    def _scale(rv, xv):
        def body(e, carry):
            m = plsc.load_gather(xv, [jnp.full((16,), e, jnp.int32)])
            for q in range(5):
                sl2 = pl.ds(q * 16, 16)
                rv[e, sl2] = rv[e, sl2] * m
            return carry
        lax.fori_loop(0, CE, body, 0)

    def _scatter(rv, dv, sem):
        pltpu.async_copy(rv, acc_sh.at[dv], sem, add=True)

    def _wait_scatter(rv, dv, sem):
        pltpu.make_async_copy(rv, acc_sh.at[dv], sem).wait()

    _load(0, src_a, dst_a, ae_a, sla)
    _wait_load(0, src_a, dst_a, ae_a, sla)
    pltpu.async_copy(ht_hbm.at[src_a], rows_a, sga)

    def _pair_body(p, carry):
        _ex_compute(src_a, dst_a, ae_a, ex_a)

        @pl.when(p > 0)
        def _():
            _wait_scatter(rows_b, dst_b, ssb)
        _load(2 * p + 1, src_b, dst_b, ae_b, slb)
        _wait_load(2 * p + 1, src_b, dst_b, ae_b, slb)
        pltpu.async_copy(ht_hbm.at[src_b], rows_b, sgb)
        pltpu.make_async_copy(ht_hbm.at[src_a], rows_a, sga).wait()
        _scale(rows_a, ex_a)
        _scatter(rows_a, dst_a, ssa)

        _ex_compute(src_b, dst_b, ae_b, ex_b)

        @pl.when(p < CPT // 2 - 1)
        def _():
            _wait_scatter(rows_a, dst_a, ssa)
            _load(2 * p + 2, src_a, dst_a, ae_a, sla)
            _wait_load(2 * p + 2, src_a, dst_a, ae_a, sla)
            pltpu.async_copy(ht_hbm.at[src_a], rows_a, sga)
        pltpu.make_async_copy(ht_hbm.at[src_b], rows_b, sgb).wait()
        _scale(rows_b, ex_b)
        _scatter(rows_b, dst_b, ssb)
        return carry

    lax.fori_loop(0, CPT // 2, _pair_body, 0)
    _wait_scatter(rows_a, dst_a, ssa)
    _wait_scatter(rows_b, dst_b, ssb)
    plsc.subcore_barrier()

    r0 = s * RPT
    pltpu.sync_copy(acc_sh.at[pl.ds(r0, RPT)], out_hbm.at[c, pl.ds(r0, RPT)])


@functools.partial(
    pl.kernel,
    out_type=jax.ShapeDtypeStruct((E2,), jnp.float32),
    mesh=_sc_mesh,
    compiler_params=_sc_params,
    scratch_types=[
        pltpu.VMEM((CE,), jnp.int32),
        pltpu.VMEM((CE,), jnp.int32),
        pltpu.VMEM((CE, DF), jnp.float32),
        pltpu.VMEM((CE, DF), jnp.float32),
        pltpu.VMEM((CE,), jnp.float32),
        pltpu.VMEM((CE,), jnp.int32),
        pltpu.VMEM((CE,), jnp.int32),
        pltpu.VMEM((CE, DF), jnp.float32),
        pltpu.VMEM((CE, DF), jnp.float32),
        pltpu.VMEM((CE,), jnp.float32),
        pltpu.VMEM((DF,), jnp.float32),
        pltpu.SemaphoreType.DMA,
        pltpu.SemaphoreType.DMA,
        pltpu.SemaphoreType.DMA,
        pltpu.SemaphoreType.DMA,
    ],
)
def _decode_pass(src_hbm, dst_hbm, uv_hbm, wp_hbm, out_hbm,
                 src_a, dst_a, urows_a, vrows_a, out_a,
                 src_b, dst_b, urows_b, vrows_b, out_b,
                 wp_v, sga, sgb, sla, slb):
    c = lax.axis_index("c")
    s = lax.axis_index("s")
    w = s * NC + c
    pltpu.sync_copy(wp_hbm, wp_v)

    def _base(j):
        return (w + j * NW) * CE

    def _load(j, sv, dv, sem):
        b = _base(j)
        pltpu.async_copy(src_hbm.at[pl.ds(b, CE)], sv, sem)
        pltpu.async_copy(dst_hbm.at[pl.ds(b, CE)], dv, sem)

    def _wait_load(j, sv, dv, sem):
        b = _base(j)
        pltpu.make_async_copy(src_hbm.at[pl.ds(b, CE)], sv, sem).wait()
        pltpu.make_async_copy(dst_hbm.at[pl.ds(b, CE)], dv, sem).wait()

    def _gather(sv, dv, uv, vv, sem):
        pltpu.async_copy(uv_hbm.at[sv], uv, sem)
        pltpu.async_copy(uv_hbm.at[dv], vv, sem)

    def _wait_gather(sv, dv, uv, vv, sem):
        pltpu.make_async_copy(uv_hbm.at[sv], uv, sem).wait()
        pltpu.make_async_copy(uv_hbm.at[dv], vv, sem).wait()

    def _compute(j, uv, vv, ov):
        lanes = lax.iota(jnp.int32, 16)
        for g in range(CE // 16):
            rowi = lanes + g * 16

            def _feat_body(k4, acc):
                for t in range(4):
                    k = k4 * 4 + t
                    ck = jnp.full((16,), k, jnp.int32)
                    uk = plsc.load_gather(uv, [rowi, ck])
                    vk = plsc.load_gather(vv, [rowi, ck + H])
                    wk = plsc.load_gather(wp_v, [ck])
                    acc = acc + jnp.maximum(uk + vk, 0.0) * wk
                return acc

            acc = lax.fori_loop(0, H // 4, _feat_body,
                                jnp.zeros((16,), jnp.float32))
            lg = acc + plsc.load_gather(
                wp_v, [jnp.full((16,), H, jnp.int32)])
            ov[pl.ds(g * 16, 16)] = 1.0 / (1.0 + jnp.exp(-lg))
        pltpu.sync_copy(ov, out_hbm.at[pl.ds(_base(j), CE)])

    _load(0, src_a, dst_a, sla)
    _wait_load(0, src_a, dst_a, sla)
    _gather(src_a, dst_a, urows_a, vrows_a, sga)

    def _pair_body(p, carry):
        _load(2 * p + 1, src_b, dst_b, slb)
        _wait_load(2 * p + 1, src_b, dst_b, slb)
        _gather(src_b, dst_b, urows_b, vrows_b, sgb)
        _wait_gather(src_a, dst_a, urows_a, vrows_a, sga)
        _compute(2 * p, urows_a, vrows_a, out_a)

        @pl.when(p < CPT // 2 - 1)
        def _():
            _load(2 * p + 2, src_a, dst_a, sla)
            _wait_load(2 * p + 2, src_a, dst_a, sla)
            _gather(src_a, dst_a, urows_a, vrows_a, sga)
        _wait_gather(src_b, dst_b, urows_b, vrows_b, sgb)
        _compute(2 * p + 1, urows_b, vrows_b, out_b)
        return carry

    lax.fori_loop(0, CPT // 2, _pair_body, 0)


BN = 2000


def _tc_encode_body(x_ref, W0_ref, b0_ref, Wc_ref, as_ref, ad_ref,
                    ht_ref, asrc_ref, adst_ref):
    z = jnp.dot(x_ref[...], W0_ref[...],
                preferred_element_type=jnp.float32) + b0_ref[...]
    h = jnp.dot(z, Wc_ref[...], preferred_element_type=jnp.float32)
    pad = jnp.concatenate(
        [h, jnp.ones((h.shape[0], 1), jnp.float32),
         jnp.zeros((h.shape[0], HP - H - 1), jnp.float32)], axis=1)
    ht_ref[...] = pad
    asrc_ref[...] = jnp.sum(h * as_ref[...], axis=1, keepdims=True)
    adst_ref[...] = jnp.sum(h * ad_ref[...], axis=1, keepdims=True)


def _tc_mid_body(acc_ref, bc_ref, Wc_ref, as_ref, ad_ref,
                 ht_ref, asrc_ref, adst_ref):
    a = acc_ref[0] + acc_ref[1]
    den = a[:, H:H + 1]
    z = jnp.maximum(a[:, :H] / (den + 1e-16) + bc_ref[...], 0.0)
    h = jnp.dot(z, Wc_ref[...], preferred_element_type=jnp.float32)
    pad = jnp.concatenate(
        [h, jnp.ones((h.shape[0], 1), jnp.float32),
         jnp.zeros((h.shape[0], HP - H - 1), jnp.float32)], axis=1)
    ht_ref[...] = pad
    asrc_ref[...] = jnp.sum(h * as_ref[...], axis=1, keepdims=True)
    adst_ref[...] = jnp.sum(h * ad_ref[...], axis=1, keepdims=True)


def _tc_final_body(acc_ref, bc_ref, W1t_ref, W1b_ref, b1_ref, uv_ref):
    a = acc_ref[0] + acc_ref[1]
    den = a[:, H:H + 1]
    z = jnp.maximum(a[:, :H] / (den + 1e-16) + bc_ref[...], 0.0)
    u = jnp.dot(z, W1t_ref[...],
                preferred_element_type=jnp.float32) + b1_ref[...]
    v = jnp.dot(z, W1b_ref[...], preferred_element_type=jnp.float32)
    uv_ref[...] = jnp.concatenate([u, v], axis=1)


def _tc_ae_body(ea_ref, Wd_ref, out_ref):
    out_ref[...] = jnp.dot(ea_ref[...], Wd_ref[...],
                           preferred_element_type=jnp.float32)


def _row_spec(bn, ncols):
    return pl.BlockSpec((bn, ncols), lambda i: (i, 0))


def _full_spec(shape):
    return pl.BlockSpec(shape, lambda i: tuple(0 for _ in shape))


def _tc_encode(x, W0, b0r, Wc, asr, adr):
    grid = (N // BN,)
    return pl.pallas_call(
        _tc_encode_body,
        grid=grid,
        in_specs=[_row_spec(BN, DF), _full_spec((DF, H)), _full_spec((1, H)),
                  _full_spec((H, H)), _full_spec((1, H)), _full_spec((1, H))],
        out_specs=[_row_spec(BN, HP),
                   _row_spec(BN, 1), _row_spec(BN, 1)],
        out_shape=[jax.ShapeDtypeStruct((N, HP), jnp.float32),
                   jax.ShapeDtypeStruct((N, 1), jnp.float32),
                   jax.ShapeDtypeStruct((N, 1), jnp.float32)],
    )(x, W0, b0r, Wc, asr, adr)


def _tc_mid(acc, bcr, Wc, asr, adr):
    grid = (N // BN,)
    return pl.pallas_call(
        _tc_mid_body,
        grid=grid,
        in_specs=[pl.BlockSpec((NC, BN, HP), lambda i: (0, i, 0)),
                  _full_spec((1, H)), _full_spec((H, H)),
                  _full_spec((1, H)), _full_spec((1, H))],
        out_specs=[_row_spec(BN, HP),
                   _row_spec(BN, 1), _row_spec(BN, 1)],
        out_shape=[jax.ShapeDtypeStruct((N, HP), jnp.float32),
                   jax.ShapeDtypeStruct((N, 1), jnp.float32),
                   jax.ShapeDtypeStruct((N, 1), jnp.float32)],
    )(acc, bcr, Wc, asr, adr)


def _tc_final(acc, bcr, W1t, W1b, b1r):
    grid = (N // BN,)
    return pl.pallas_call(
        _tc_final_body,
        grid=grid,
        in_specs=[pl.BlockSpec((NC, BN, HP), lambda i: (0, i, 0)),
                  _full_spec((1, H)), _full_spec((H, H)),
                  _full_spec((H, H)), _full_spec((1, H))],
        out_specs=_row_spec(BN, DF),
        out_shape=jax.ShapeDtypeStruct((N, DF), jnp.float32),
    )(acc, bcr, W1t, W1b, b1r)


def _tc_ae(ea128, Wd):
    R = E // 8
    BR = R // 5
    return pl.pallas_call(
        _tc_ae_body,
        grid=(5,),
        in_specs=[_row_spec(BR, DF), _full_spec((DF, 16))],
        out_specs=_row_spec(BR, 16),
        out_shape=jax.ShapeDtypeStruct((R, 16), jnp.float32),
    )(ea128, Wd)


def kernel(x, edge_index, edge_attr, W0, b0, Wc1, as1, ad1, We1, ae1, bc1,
           Wc2, as2, ad2, We2, ae2, bc2, W1, b1, W2, b2):
    src = edge_index[0]
    dst = edge_index[1]

    b0r = b0.reshape(1, H)
    bc1r = bc1.reshape(1, H)
    bc2r = bc2.reshape(1, H)
    b1r = b1.reshape(1, H)
    as1r = as1.reshape(1, H)
    ad1r = ad1.reshape(1, H)
    as2r = as2.reshape(1, H)
    ad2r = ad2.reshape(1, H)
    w1e = We1 @ ae1
    w2e = We2 @ ae2
    rows = jnp.arange(DF)
    arow = rows // DE
    krow = rows % DE
    Wd = jnp.zeros((DF, 16), jnp.float32)
    Wd = Wd.at[rows, arow].set(w1e[krow])
    Wd = Wd.at[rows, 8 + arow].set(w2e[krow])
    ea128 = edge_attr.reshape(E // 8, DF)
    W1t = W1[:H]
    W1b = W1[H:]
    wp = jnp.concatenate(
        [W2.reshape(H), b2.reshape(1), jnp.zeros((DF - H - 1,), jnp.float32)])

    ht1, asrc1, adst1 = _tc_encode(x, W0, b0r, Wc1, as1r, ad1r)
    aeo = _tc_ae(ea128, Wd)
    ae1v = aeo[:, :8].reshape(E)
    ae2v = aeo[:, 8:].reshape(E)

    npad = E2 - E
    srcp = jnp.concatenate([src, jnp.zeros((npad,), jnp.int32)])
    dstg = jnp.concatenate([dst, jnp.full((npad,), NP - 1, jnp.int32)])
    dstd = jnp.concatenate([dst, jnp.zeros((npad,), jnp.int32)])
    zpad = jnp.zeros((npad,), jnp.float32)
    ae1p = jnp.concatenate([ae1v, zpad])
    ae2p = jnp.concatenate([ae2v, zpad])
    zro = jnp.zeros((RPT, HP), jnp.float32)

    acc1 = _gat_edge_pass(srcp, dstg, ae1p, asrc1.reshape(N),
                          adst1.reshape(N), ht1, zro)
    ht2, asrc2, adst2 = _tc_mid(acc1, bc1r, Wc2, as2r, ad2r)
    acc2 = _gat_edge_pass(srcp, dstg, ae2p, asrc2.reshape(N),
                          adst2.reshape(N), ht2, zro)
    uv = _tc_final(acc2, bc2r, W1t, W1b, b1r)
    return _decode_pass(srcp, dstd, uv, wp)[:E]


# speedup vs baseline: 1.1812x; 1.1812x over previous
import functools

import jax
import jax.numpy as jnp
from jax import lax
from jax.experimental import pallas as pl
from jax.experimental.pallas import tpu as pltpu
from jax.experimental.pallas import tpu_sc as plsc

N = 10000
E = 320000
DF = 128
DE = 16
H = 64
HP = 80
NC = 2
NS = 16
NW = NC * NS
CE = 128
CPT = 80
E2 = NW * CPT * CE
NP = 10240
RPT = NP // NS

_sc_mesh = plsc.VectorSubcoreMesh(core_axis_name="c", subcore_axis_name="s")
_sc_params = pltpu.CompilerParams(needs_layout_passes=False,
                                  use_tc_tiling_on_sc=False)


@functools.partial(
    pl.kernel,
    out_type=jax.ShapeDtypeStruct((NC, NP, HP), jnp.float32),
    mesh=_sc_mesh,
    compiler_params=_sc_params,
    scratch_types=[
        pltpu.VMEM((N,), jnp.float32),
        pltpu.VMEM((N,), jnp.float32),
        pltpu.VMEM((CE,), jnp.int32),
        pltpu.VMEM((CE,), jnp.int32),
        pltpu.VMEM((CE,), jnp.float32),
        pltpu.VMEM((CE,), jnp.float32),
        pltpu.VMEM((CE, HP), jnp.float32),
        pltpu.VMEM((CE,), jnp.int32),
        pltpu.VMEM((CE,), jnp.int32),
        pltpu.VMEM((CE,), jnp.float32),
        pltpu.VMEM((CE,), jnp.float32),
        pltpu.VMEM((CE, HP), jnp.float32),
        pltpu.VMEM_SHARED((NP, HP), jnp.float32),
        pltpu.SemaphoreType.DMA,
        pltpu.SemaphoreType.DMA,
        pltpu.SemaphoreType.DMA,
        pltpu.SemaphoreType.DMA,
        pltpu.SemaphoreType.DMA,
        pltpu.SemaphoreType.DMA,
    ],
)
def _gat_edge_pass(src_hbm, dst_hbm, ae_hbm, asrc_hbm, adst_hbm, ht_hbm,
                   zro_hbm, out_hbm, asrc_v, adst_v,
                   src_a, dst_a, ae_a, ex_a, rows_a,
                   src_b, dst_b, ae_b, ex_b, rows_b,
                   acc_sh, sga, sgb, ssa, ssb, sla, slb):
    c = lax.axis_index("c")
    s = lax.axis_index("s")
    w = s * NC + c

    pltpu.sync_copy(zro_hbm, acc_sh.at[pl.ds(s * RPT, RPT)])

    pltpu.sync_copy(asrc_hbm, asrc_v)
    pltpu.sync_copy(adst_hbm, adst_v)
    plsc.subcore_barrier()

    def _base(j):
        return (w + j * NW) * CE

    def _load(j, sv, dv, av, sem):
        b = _base(j)
        pltpu.async_copy(src_hbm.at[pl.ds(b, CE)], sv, sem)
        pltpu.async_copy(dst_hbm.at[pl.ds(b, CE)], dv, sem)
        pltpu.async_copy(ae_hbm.at[pl.ds(b, CE)], av, sem)

    def _wait_load(j, sv, dv, av, sem):
        b = _base(j)
        pltpu.make_async_copy(src_hbm.at[pl.ds(b, CE)], sv, sem).wait()
        pltpu.make_async_copy(dst_hbm.at[pl.ds(b, CE)], dv, sem).wait()
        pltpu.make_async_copy(ae_hbm.at[pl.ds(b, CE)], av, sem).wait()

    def _ex_compute(sv, dv, av, xv):
        for g in range(CE // 16):
            sl = pl.ds(g * 16, 16)
            lg = (plsc.load_gather(asrc_v, [sv[sl]])
                  + plsc.load_gather(adst_v, [dv[sl]]) + av[sl])
            lg = jnp.where(lg > 0, lg, 0.2 * lg)
            xv[sl] = jnp.exp(lg)

    def _scale(rv, xv):
        def body(e, carry):
            m = plsc.load_gather(xv, [jnp.full((16,), e, jnp.int32)])
            for q in range(5):
                sl2 = pl.ds(q * 16, 16)
                rv[e, sl2] = rv[e, sl2] * m
            return carry
        lax.fori_loop(0, CE, body, 0)

    def _scatter(rv, dv, sem):
        pltpu.async_copy(rv, acc_sh.at[dv], sem, add=True)

    def _wait_scatter(rv, dv, sem):
        pltpu.make_async_copy(rv, acc_sh.at[dv], sem).wait()

    _load(0, src_a, dst_a, ae_a, sla)
    _wait_load(0, src_a, dst_a, ae_a, sla)
    pltpu.async_copy(ht_hbm.at[src_a], rows_a, sga)

    def _pair_body(p, carry):
        _ex_compute(src_a, dst_a, ae_a, ex_a)

        @pl.when(p > 0)
        def _():
            _wait_scatter(rows_b, dst_b, ssb)
        _load(2 * p + 1, src_b, dst_b, ae_b, slb)
        _wait_load(2 * p + 1, src_b, dst_b, ae_b, slb)
        pltpu.async_copy(ht_hbm.at[src_b], rows_b, sgb)
        pltpu.make_async_copy(ht_hbm.at[src_a], rows_a, sga).wait()
        _scale(rows_a, ex_a)
        _scatter(rows_a, dst_a, ssa)

        _ex_compute(src_b, dst_b, ae_b, ex_b)

        @pl.when(p < CPT // 2 - 1)
        def _():
            _wait_scatter(rows_a, dst_a, ssa)
            _load(2 * p + 2, src_a, dst_a, ae_a, sla)
            _wait_load(2 * p + 2, src_a, dst_a, ae_a, sla)
            pltpu.async_copy(ht_hbm.at[src_a], rows_a, sga)
        pltpu.make_async_copy(ht_hbm.at[src_b], rows_b, sgb).wait()
        _scale(rows_b, ex_b)
        _scatter(rows_b, dst_b, ssb)
        return carry

    lax.fori_loop(0, CPT // 2, _pair_body, 0)
    _wait_scatter(rows_a, dst_a, ssa)
    _wait_scatter(rows_b, dst_b, ssb)
    plsc.subcore_barrier()

    r0 = s * RPT
    pltpu.sync_copy(acc_sh.at[pl.ds(r0, RPT)], out_hbm.at[c, pl.ds(r0, RPT)])


@functools.partial(
    pl.kernel,
    out_type=jax.ShapeDtypeStruct((E2,), jnp.float32),
    mesh=_sc_mesh,
    compiler_params=_sc_params,
    scratch_types=[
        pltpu.VMEM((CE,), jnp.int32),
        pltpu.VMEM((CE,), jnp.int32),
        pltpu.VMEM((CE, H), jnp.float32),
        pltpu.VMEM((CE, H), jnp.float32),
        pltpu.VMEM((CE,), jnp.float32),
        pltpu.VMEM((CE,), jnp.int32),
        pltpu.VMEM((CE,), jnp.int32),
        pltpu.VMEM((CE, H), jnp.float32),
        pltpu.VMEM((CE, H), jnp.float32),
        pltpu.VMEM((CE,), jnp.float32),
        pltpu.VMEM((80,), jnp.float32),
        pltpu.SemaphoreType.DMA,
        pltpu.SemaphoreType.DMA,
        pltpu.SemaphoreType.DMA,
        pltpu.SemaphoreType.DMA,
    ],
)
def _decode_pass(src_hbm, dst_hbm, u_hbm, v_hbm, wp_hbm, out_hbm,
                 src_a, dst_a, urows_a, vrows_a, out_a,
                 src_b, dst_b, urows_b, vrows_b, out_b,
                 wp_v, sga, sgb, sla, slb):
    c = lax.axis_index("c")
    s = lax.axis_index("s")
    w = s * NC + c
    pltpu.sync_copy(wp_hbm, wp_v)

    def _base(j):
        return (w + j * NW) * CE

    def _load(j, sv, dv, sem):
        b = _base(j)
        pltpu.async_copy(src_hbm.at[pl.ds(b, CE)], sv, sem)
        pltpu.async_copy(dst_hbm.at[pl.ds(b, CE)], dv, sem)

    def _wait_load(j, sv, dv, sem):
        b = _base(j)
        pltpu.make_async_copy(src_hbm.at[pl.ds(b, CE)], sv, sem).wait()
        pltpu.make_async_copy(dst_hbm.at[pl.ds(b, CE)], dv, sem).wait()

    def _gather(sv, dv, uv, vv, sem):
        pltpu.async_copy(u_hbm.at[sv], uv, sem)
        pltpu.async_copy(v_hbm.at[dv], vv, sem)

    def _wait_gather(sv, dv, uv, vv, sem):
        pltpu.make_async_copy(u_hbm.at[sv], uv, sem).wait()
        pltpu.make_async_copy(v_hbm.at[dv], vv, sem).wait()

    def _compute(j, uv, vv, ov):
        lanes = lax.iota(jnp.int32, 16)
        for g in range(CE // 16):
            rowi = lanes + g * 16

            def _feat_body(k4, acc):
                for t in range(4):
                    k = k4 * 4 + t
                    ck = jnp.full((16,), k, jnp.int32)
                    uk = plsc.load_gather(uv, [rowi, ck])
                    vk = plsc.load_gather(vv, [rowi, ck])
                    wk = plsc.load_gather(wp_v, [ck])
                    acc = acc + jnp.maximum(uk + vk, 0.0) * wk
                return acc

            acc = lax.fori_loop(0, H // 4, _feat_body,
                                jnp.zeros((16,), jnp.float32))
            lg = acc + plsc.load_gather(
                wp_v, [jnp.full((16,), H, jnp.int32)])
            ov[pl.ds(g * 16, 16)] = 1.0 / (1.0 + jnp.exp(-lg))
        pltpu.sync_copy(ov, out_hbm.at[pl.ds(_base(j), CE)])

    _load(0, src_a, dst_a, sla)
    _wait_load(0, src_a, dst_a, sla)
    _gather(src_a, dst_a, urows_a, vrows_a, sga)

    def _pair_body(p, carry):
        _load(2 * p + 1, src_b, dst_b, slb)
        _wait_load(2 * p + 1, src_b, dst_b, slb)
        _gather(src_b, dst_b, urows_b, vrows_b, sgb)
        _wait_gather(src_a, dst_a, urows_a, vrows_a, sga)
        _compute(2 * p, urows_a, vrows_a, out_a)

        @pl.when(p < CPT // 2 - 1)
        def _():
            _load(2 * p + 2, src_a, dst_a, sla)
            _wait_load(2 * p + 2, src_a, dst_a, sla)
            _gather(src_a, dst_a, urows_a, vrows_a, sga)
        _wait_gather(src_b, dst_b, urows_b, vrows_b, sgb)
        _compute(2 * p + 1, urows_b, vrows_b, out_b)
        return carry

    lax.fori_loop(0, CPT // 2, _pair_body, 0)


BN = 2000


def _tc_encode_body(x_ref, W0_ref, b0_ref, Wc_ref, as_ref, ad_ref,
                    ht_ref, asrc_ref, adst_ref):
    z = jnp.dot(x_ref[...], W0_ref[...],
                preferred_element_type=jnp.float32) + b0_ref[...]
    h = jnp.dot(z, Wc_ref[...], preferred_element_type=jnp.float32)
    pad = jnp.concatenate(
        [h, jnp.ones((h.shape[0], 1), jnp.float32),
         jnp.zeros((h.shape[0], HP - H - 1), jnp.float32)], axis=1)
    ht_ref[...] = pad
    asrc_ref[...] = jnp.sum(h * as_ref[...], axis=1, keepdims=True)
    adst_ref[...] = jnp.sum(h * ad_ref[...], axis=1, keepdims=True)


def _tc_mid_body(acc_ref, bc_ref, Wc_ref, as_ref, ad_ref,
                 ht_ref, asrc_ref, adst_ref):
    a = acc_ref[0] + acc_ref[1]
    den = a[:, H:H + 1]
    z = jnp.maximum(a[:, :H] / (den + 1e-16) + bc_ref[...], 0.0)
    h = jnp.dot(z, Wc_ref[...], preferred_element_type=jnp.float32)
    pad = jnp.concatenate(
        [h, jnp.ones((h.shape[0], 1), jnp.float32),
         jnp.zeros((h.shape[0], HP - H - 1), jnp.float32)], axis=1)
    ht_ref[...] = pad
    asrc_ref[...] = jnp.sum(h * as_ref[...], axis=1, keepdims=True)
    adst_ref[...] = jnp.sum(h * ad_ref[...], axis=1, keepdims=True)


def _tc_final_body(acc_ref, bc_ref, W1t_ref, W1b_ref, b1_ref, u_ref, v_ref):
    a = acc_ref[0] + acc_ref[1]
    den = a[:, H:H + 1]
    z = jnp.maximum(a[:, :H] / (den + 1e-16) + bc_ref[...], 0.0)
    u_ref[...] = jnp.dot(z, W1t_ref[...],
                         preferred_element_type=jnp.float32) + b1_ref[...]
    v_ref[...] = jnp.dot(z, W1b_ref[...], preferred_element_type=jnp.float32)


def _tc_ae_body(ea_ref, Wd_ref, out_ref):
    out_ref[...] = jnp.dot(ea_ref[...], Wd_ref[...],
                           preferred_element_type=jnp.float32)


def _row_spec(bn, ncols):
    return pl.BlockSpec((bn, ncols), lambda i: (i, 0))


def _full_spec(shape):
    return pl.BlockSpec(shape, lambda i: tuple(0 for _ in shape))


def _tc_encode(x, W0, b0r, Wc, asr, adr):
    grid = (N // BN,)
    return pl.pallas_call(
        _tc_encode_body,
        grid=grid,
        in_specs=[_row_spec(BN, DF), _full_spec((DF, H)), _full_spec((1, H)),
                  _full_spec((H, H)), _full_spec((1, H)), _full_spec((1, H))],
        out_specs=[_row_spec(BN, HP),
                   _row_spec(BN, 1), _row_spec(BN, 1)],
        out_shape=[jax.ShapeDtypeStruct((N, HP), jnp.float32),
                   jax.ShapeDtypeStruct((N, 1), jnp.float32),
                   jax.ShapeDtypeStruct((N, 1), jnp.float32)],
    )(x, W0, b0r, Wc, asr, adr)


def _tc_mid(acc, bcr, Wc, asr, adr):
    grid = (N // BN,)
    return pl.pallas_call(
        _tc_mid_body,
        grid=grid,
        in_specs=[pl.BlockSpec((NC, BN, HP), lambda i: (0, i, 0)),
                  _full_spec((1, H)), _full_spec((H, H)),
                  _full_spec((1, H)), _full_spec((1, H))],
        out_specs=[_row_spec(BN, HP),
                   _row_spec(BN, 1), _row_spec(BN, 1)],
        out_shape=[jax.ShapeDtypeStruct((N, HP), jnp.float32),
                   jax.ShapeDtypeStruct((N, 1), jnp.float32),
                   jax.ShapeDtypeStruct((N, 1), jnp.float32)],
    )(acc, bcr, Wc, asr, adr)


def _tc_final(acc, bcr, W1t, W1b, b1r):
    grid = (N // BN,)
    return pl.pallas_call(
        _tc_final_body,
        grid=grid,
        in_specs=[pl.BlockSpec((NC, BN, HP), lambda i: (0, i, 0)),
                  _full_spec((1, H)), _full_spec((H, H)),
                  _full_spec((H, H)), _full_spec((1, H))],
        out_specs=[_row_spec(BN, H), _row_spec(BN, H)],
        out_shape=[jax.ShapeDtypeStruct((N, H), jnp.float32),
                   jax.ShapeDtypeStruct((N, H), jnp.float32)],
    )(acc, bcr, W1t, W1b, b1r)


def _tc_ae(ea128, Wd):
    R = E // 8
    BR = R // 5
    return pl.pallas_call(
        _tc_ae_body,
        grid=(5,),
        in_specs=[_row_spec(BR, DF), _full_spec((DF, 16))],
        out_specs=_row_spec(BR, 16),
        out_shape=jax.ShapeDtypeStruct((R, 16), jnp.float32),
    )(ea128, Wd)


def kernel(x, edge_index, edge_attr, W0, b0, Wc1, as1, ad1, We1, ae1, bc1,
           Wc2, as2, ad2, We2, ae2, bc2, W1, b1, W2, b2):
    src = edge_index[0]
    dst = edge_index[1]

    b0r = b0.reshape(1, H)
    bc1r = bc1.reshape(1, H)
    bc2r = bc2.reshape(1, H)
    b1r = b1.reshape(1, H)
    as1r = as1.reshape(1, H)
    ad1r = ad1.reshape(1, H)
    as2r = as2.reshape(1, H)
    ad2r = ad2.reshape(1, H)
    w1e = We1 @ ae1
    w2e = We2 @ ae2
    rows = jnp.arange(DF)
    arow = rows // DE
    krow = rows % DE
    Wd = jnp.zeros((DF, 16), jnp.float32)
    Wd = Wd.at[rows, arow].set(w1e[krow])
    Wd = Wd.at[rows, 8 + arow].set(w2e[krow])
    ea128 = edge_attr.reshape(E // 8, DF)
    W1t = W1[:H]
    W1b = W1[H:]
    wp = jnp.concatenate(
        [W2.reshape(H), b2.reshape(1), jnp.zeros((15,), jnp.float32)])

    ht1, asrc1, adst1 = _tc_encode(x, W0, b0r, Wc1, as1r, ad1r)
    aeo = _tc_ae(ea128, Wd)
    ae1v = aeo[:, :8].reshape(E)
    ae2v = aeo[:, 8:].reshape(E)

    npad = E2 - E
    srcp = jnp.concatenate([src, jnp.zeros((npad,), jnp.int32)])
    dstg = jnp.concatenate([dst, jnp.full((npad,), NP - 1, jnp.int32)])
    dstd = jnp.concatenate([dst, jnp.zeros((npad,), jnp.int32)])
    zpad = jnp.zeros((npad,), jnp.float32)
    ae1p = jnp.concatenate([ae1v, zpad])
    ae2p = jnp.concatenate([ae2v, zpad])
    zro = jnp.zeros((RPT, HP), jnp.float32)

    acc1 = _gat_edge_pass(srcp, dstg, ae1p, asrc1.reshape(N),
                          adst1.reshape(N), ht1, zro)
    ht2, asrc2, adst2 = _tc_mid(acc1, bc1r, Wc2, as2r, ad2r)
    acc2 = _gat_edge_pass(srcp, dstg, ae2p, asrc2.reshape(N),
                          adst2.reshape(N), ht2, zro)
    u, v = _tc_final(acc2, bc2r, W1t, W1b, b1r)
    return _decode_pass(srcp, dstd, u, v, wp)[:E]


# speedup vs baseline: 1.4872x; 1.2590x over previous
import functools

import jax
import jax.numpy as jnp
from jax import lax
from jax.experimental import pallas as pl
from jax.experimental.pallas import tpu as pltpu
from jax.experimental.pallas import tpu_sc as plsc

N = 10000
E = 320000
DF = 128
DE = 16
H = 64
HP = 80
NC = 2
NS = 16
NW = NC * NS
CE = 128
CPT = 80
E2 = NW * CPT * CE
NP = 10240
RPT = NP // NS

_sc_mesh = plsc.VectorSubcoreMesh(core_axis_name="c", subcore_axis_name="s")
_sc_params = pltpu.CompilerParams(needs_layout_passes=False,
                                  use_tc_tiling_on_sc=False)


@functools.partial(
    pl.kernel,
    out_type=jax.ShapeDtypeStruct((NC, NP, HP), jnp.float32),
    mesh=_sc_mesh,
    compiler_params=_sc_params,
    scratch_types=[
        pltpu.VMEM((N,), jnp.float32),
        pltpu.VMEM((N,), jnp.float32),
        pltpu.VMEM((CE,), jnp.int32),
        pltpu.VMEM((CE,), jnp.int32),
        pltpu.VMEM((CE,), jnp.float32),
        pltpu.VMEM((CE,), jnp.float32),
        pltpu.VMEM((CE, HP), jnp.float32),
        pltpu.VMEM((CE,), jnp.int32),
        pltpu.VMEM((CE,), jnp.int32),
        pltpu.VMEM((CE,), jnp.float32),
        pltpu.VMEM((CE,), jnp.float32),
        pltpu.VMEM((CE, HP), jnp.float32),
        pltpu.VMEM_SHARED((NP, HP), jnp.float32),
        pltpu.SemaphoreType.DMA,
        pltpu.SemaphoreType.DMA,
        pltpu.SemaphoreType.DMA,
        pltpu.SemaphoreType.DMA,
        pltpu.SemaphoreType.DMA,
        pltpu.SemaphoreType.DMA,
    ],
)
def _gat_edge_pass(src_hbm, dst_hbm, ae_hbm, asrc_hbm, adst_hbm, ht_hbm,
                   zro_hbm, out_hbm, asrc_v, adst_v,
                   src_a, dst_a, ae_a, ex_a, rows_a,
                   src_b, dst_b, ae_b, ex_b, rows_b,
                   acc_sh, sga, sgb, ssa, ssb, sla, slb):
    c = lax.axis_index("c")
    s = lax.axis_index("s")
    w = s * NC + c

    pltpu.sync_copy(zro_hbm, acc_sh.at[pl.ds(s * RPT, RPT)])

    pltpu.sync_copy(asrc_hbm, asrc_v)
    pltpu.sync_copy(adst_hbm, adst_v)
    plsc.subcore_barrier()

    def _base(j):
        return (w + j * NW) * CE

    def _load(j, sv, dv, av, sem):
        b = _base(j)
        pltpu.async_copy(src_hbm.at[pl.ds(b, CE)], sv, sem)
        pltpu.async_copy(dst_hbm.at[pl.ds(b, CE)], dv, sem)
        pltpu.async_copy(ae_hbm.at[pl.ds(b, CE)], av, sem)

    def _wait_load(j, sv, dv, av, sem):
        b = _base(j)
        pltpu.make_async_copy(src_hbm.at[pl.ds(b, CE)], sv, sem).wait()
        pltpu.make_async_copy(dst_hbm.at[pl.ds(b, CE)], dv, sem).wait()
        pltpu.make_async_copy(ae_hbm.at[pl.ds(b, CE)], av, sem).wait()

    def _ex_compute(sv, dv, av, xv):
        for g in range(CE // 16):
            sl = pl.ds(g * 16, 16)
            lg = (plsc.load_gather(asrc_v, [sv[sl]])
                  + plsc.load_gather(adst_v, [dv[sl]]) + av[sl])
            lg = jnp.where(lg > 0, lg, 0.2 * lg)
            xv[sl] = jnp.exp(lg)

    def _scale(rv, xv):
        def body(e, carry):
            m = plsc.load_gather(xv, [jnp.full((16,), e, jnp.int32)])
            for q in range(5):
                sl2 = pl.ds(q * 16, 16)
                rv[e, sl2] = rv[e, sl2] * m
            return carry
        lax.fori_loop(0, CE, body, 0)

    def _scatter(rv, dv, sem):
        pltpu.async_copy(rv, acc_sh.at[dv], sem, add=True)

    def _wait_scatter(rv, dv, sem):
        pltpu.make_async_copy(rv, acc_sh.at[dv], sem).wait()

    _load(0, src_a, dst_a, ae_a, sla)
    _wait_load(0, src_a, dst_a, ae_a, sla)
    pltpu.async_copy(ht_hbm.at[src_a], rows_a, sga)

    def _pair_body(p, carry):
        _ex_compute(src_a, dst_a, ae_a, ex_a)

        @pl.when(p > 0)
        def _():
            _wait_scatter(rows_b, dst_b, ssb)
        _load(2 * p + 1, src_b, dst_b, ae_b, slb)
        _wait_load(2 * p + 1, src_b, dst_b, ae_b, slb)
        pltpu.async_copy(ht_hbm.at[src_b], rows_b, sgb)
        pltpu.make_async_copy(ht_hbm.at[src_a], rows_a, sga).wait()
        _scale(rows_a, ex_a)
        _scatter(rows_a, dst_a, ssa)

        _ex_compute(src_b, dst_b, ae_b, ex_b)

        @pl.when(p < CPT // 2 - 1)
        def _():
            _wait_scatter(rows_a, dst_a, ssa)
            _load(2 * p + 2, src_a, dst_a, ae_a, sla)
            _wait_load(2 * p + 2, src_a, dst_a, ae_a, sla)
            pltpu.async_copy(ht_hbm.at[src_a], rows_a, sga)
        pltpu.make_async_copy(ht_hbm.at[src_b], rows_b, sgb).wait()
        _scale(rows_b, ex_b)
        _scatter(rows_b, dst_b, ssb)
        return carry

    lax.fori_loop(0, CPT // 2, _pair_body, 0)
    _wait_scatter(rows_a, dst_a, ssa)
    _wait_scatter(rows_b, dst_b, ssb)
    plsc.subcore_barrier()

    r0 = s * RPT
    pltpu.sync_copy(acc_sh.at[pl.ds(r0, RPT)], out_hbm.at[c, pl.ds(r0, RPT)])


CED = 2048
EHALF = E2 // 2


@functools.partial(
    pl.kernel,
    out_type=jax.ShapeDtypeStruct((NC, NS, EHALF), jnp.float32),
    mesh=_sc_mesh,
    compiler_params=_sc_params,
    scratch_types=[
        pltpu.VMEM((4 * N,), jnp.float32),
        pltpu.VMEM((4 * N,), jnp.float32),
        pltpu.VMEM((CED,), jnp.int32),
        pltpu.VMEM((CED,), jnp.int32),
        pltpu.VMEM((CED,), jnp.float32),
        pltpu.VMEM((CED,), jnp.int32),
        pltpu.VMEM((CED,), jnp.int32),
        pltpu.VMEM((CED,), jnp.float32),
        pltpu.VMEM((80,), jnp.float32),
        pltpu.SemaphoreType.DMA,
        pltpu.SemaphoreType.DMA,
        pltpu.SemaphoreType.DMA,
        pltpu.SemaphoreType.DMA,
    ],
)
def _decode_pass(src_hbm, dst_hbm, ug_hbm, vg_hbm, wp_hbm, out_hbm,
                 u_t, v_t, src_a, dst_a, out_a, src_b, dst_b, out_b,
                 wp_v, sla, slb, ssa, ssb):
    c = lax.axis_index("c")
    s = lax.axis_index("s")
    pltpu.sync_copy(wp_hbm, wp_v)
    pltpu.sync_copy(ug_hbm.at[pl.ds(s * 4 * N, 4 * N)], u_t)
    pltpu.sync_copy(vg_hbm.at[pl.ds(s * 4 * N, 4 * N)], v_t)
    wks = [plsc.load_gather(wp_v, [jnp.full((16,), 0, jnp.int32) + s * 4 + k])
           for k in range(4)]

    def _eoff(j):
        return c * EHALF + j * CED

    def _load(j, sv, dv, sem):
        b = _eoff(j)
        pltpu.async_copy(src_hbm.at[pl.ds(b, CED)], sv, sem)
        pltpu.async_copy(dst_hbm.at[pl.ds(b, CED)], dv, sem)

    def _wait_load(j, sv, dv, sem):
        b = _eoff(j)
        pltpu.make_async_copy(src_hbm.at[pl.ds(b, CED)], sv, sem).wait()
        pltpu.make_async_copy(dst_hbm.at[pl.ds(b, CED)], dv, sem).wait()

    def _compute(sv, dv, ov):
        def grp(i, carry):
            sl = pl.ds(i * 16, 16)
            si = sv[sl]
            di = dv[sl]
            acc = jnp.zeros((16,), jnp.float32)
            si4 = si * 4
            di4 = di * 4
            for k in range(4):
                uk = plsc.load_gather(u_t, [si4 + k])
                vk = plsc.load_gather(v_t, [di4 + k])
                acc = acc + jnp.maximum(uk + vk, 0.0) * wks[k]
            ov[sl] = acc
            return carry
        lax.fori_loop(0, CED // 16, grp, 0, unroll=2)

    def _store(j, ov, sem):
        pltpu.async_copy(ov, out_hbm.at[c, s, pl.ds(j * CED, CED)], sem)

    def _wait_store(j, ov, sem):
        pltpu.make_async_copy(ov, out_hbm.at[c, s, pl.ds(j * CED, CED)],
                              sem).wait()

    nch = EHALF // CED

    _load(0, src_a, dst_a, sla)
    _wait_load(0, src_a, dst_a, sla)

    def _pair_body(p, carry):
        _load(2 * p + 1, src_b, dst_b, slb)

        @pl.when(p > 0)
        def _():
            _wait_store(2 * p - 2, out_a, ssa)
        _compute(src_a, dst_a, out_a)
        _store(2 * p, out_a, ssa)
        _wait_load(2 * p + 1, src_b, dst_b, slb)

        @pl.when(p < nch // 2 - 1)
        def _():
            _load(2 * p + 2, src_a, dst_a, sla)

        @pl.when(p > 0)
        def _():
            _wait_store(2 * p - 1, out_b, ssb)
        _compute(src_b, dst_b, out_b)
        _store(2 * p + 1, out_b, ssb)

        @pl.when(p < nch // 2 - 1)
        def _():
            _wait_load(2 * p + 2, src_a, dst_a, sla)
        return carry

    lax.fori_loop(0, nch // 2, _pair_body, 0)
    _wait_store(nch - 2, out_a, ssa)
    _wait_store(nch - 1, out_b, ssb)


BN = 2000


def _tc_encode_body(x_ref, W0_ref, b0_ref, Wc_ref, as_ref, ad_ref,
                    ht_ref, asrc_ref, adst_ref):
    z = jnp.dot(x_ref[...], W0_ref[...],
                preferred_element_type=jnp.float32) + b0_ref[...]
    h = jnp.dot(z, Wc_ref[...], preferred_element_type=jnp.float32)
    pad = jnp.concatenate(
        [h, jnp.ones((h.shape[0], 1), jnp.float32),
         jnp.zeros((h.shape[0], HP - H - 1), jnp.float32)], axis=1)
    ht_ref[...] = pad
    asrc_ref[...] = jnp.sum(h * as_ref[...], axis=1, keepdims=True)
    adst_ref[...] = jnp.sum(h * ad_ref[...], axis=1, keepdims=True)


def _tc_mid_body(acc_ref, bc_ref, Wc_ref, as_ref, ad_ref,
                 ht_ref, asrc_ref, adst_ref):
    a = acc_ref[0] + acc_ref[1]
    den = a[:, H:H + 1]
    z = jnp.maximum(a[:, :H] / (den + 1e-16) + bc_ref[...], 0.0)
    h = jnp.dot(z, Wc_ref[...], preferred_element_type=jnp.float32)
    pad = jnp.concatenate(
        [h, jnp.ones((h.shape[0], 1), jnp.float32),
         jnp.zeros((h.shape[0], HP - H - 1), jnp.float32)], axis=1)
    ht_ref[...] = pad
    asrc_ref[...] = jnp.sum(h * as_ref[...], axis=1, keepdims=True)
    adst_ref[...] = jnp.sum(h * ad_ref[...], axis=1, keepdims=True)


def _tc_final_body(acc_ref, bc_ref, W1t_ref, W1b_ref, b1_ref, u_ref, v_ref):
    a = acc_ref[0] + acc_ref[1]
    den = a[:, H:H + 1]
    z = jnp.maximum(a[:, :H] / (den + 1e-16) + bc_ref[...], 0.0)
    u_ref[...] = jnp.dot(z, W1t_ref[...],
                         preferred_element_type=jnp.float32) + b1_ref[...]
    v_ref[...] = jnp.dot(z, W1b_ref[...], preferred_element_type=jnp.float32)


def _tc_gsum_body(p_ref, b2_ref, out_ref):
    out_ref[...] = jax.nn.sigmoid(jnp.concatenate(
        [jnp.sum(p_ref[0], axis=0, keepdims=True),
         jnp.sum(p_ref[1], axis=0, keepdims=True)], axis=0) + b2_ref[...])


def _tc_gsum(partials, b2r):
    BE2 = EHALF // 10
    return pl.pallas_call(
        _tc_gsum_body,
        grid=(10,),
        in_specs=[pl.BlockSpec((NC, NS, BE2), lambda i: (0, 0, i)),
                  _full_spec((1, 1))],
        out_specs=pl.BlockSpec((NC, BE2), lambda i: (0, i)),
        out_shape=jax.ShapeDtypeStruct((NC, EHALF), jnp.float32),
    )(partials, b2r)


def _tc_ae_body(ea_ref, Wd_ref, out_ref):
    out_ref[...] = jnp.dot(ea_ref[...], Wd_ref[...],
                           preferred_element_type=jnp.float32)


def _row_spec(bn, ncols):
    return pl.BlockSpec((bn, ncols), lambda i: (i, 0))


def _full_spec(shape):
    return pl.BlockSpec(shape, lambda i: tuple(0 for _ in shape))


def _tc_encode(x, W0, b0r, Wc, asr, adr):
    grid = (N // BN,)
    return pl.pallas_call(
        _tc_encode_body,
        grid=grid,
        in_specs=[_row_spec(BN, DF), _full_spec((DF, H)), _full_spec((1, H)),
                  _full_spec((H, H)), _full_spec((1, H)), _full_spec((1, H))],
        out_specs=[_row_spec(BN, HP),
                   _row_spec(BN, 1), _row_spec(BN, 1)],
        out_shape=[jax.ShapeDtypeStruct((N, HP), jnp.float32),
                   jax.ShapeDtypeStruct((N, 1), jnp.float32),
                   jax.ShapeDtypeStruct((N, 1), jnp.float32)],
    )(x, W0, b0r, Wc, asr, adr)


def _tc_mid(acc, bcr, Wc, asr, adr):
    grid = (N // BN,)
    return pl.pallas_call(
        _tc_mid_body,
        grid=grid,
        in_specs=[pl.BlockSpec((NC, BN, HP), lambda i: (0, i, 0)),
                  _full_spec((1, H)), _full_spec((H, H)),
                  _full_spec((1, H)), _full_spec((1, H))],
        out_specs=[_row_spec(BN, HP),
                   _row_spec(BN, 1), _row_spec(BN, 1)],
        out_shape=[jax.ShapeDtypeStruct((N, HP), jnp.float32),
                   jax.ShapeDtypeStruct((N, 1), jnp.float32),
                   jax.ShapeDtypeStruct((N, 1), jnp.float32)],
    )(acc, bcr, Wc, asr, adr)


def _tc_final(acc, bcr, W1t, W1b, b1r):
    grid = (N // BN,)
    return pl.pallas_call(
        _tc_final_body,
        grid=grid,
        in_specs=[pl.BlockSpec((NC, BN, HP), lambda i: (0, i, 0)),
                  _full_spec((1, H)), _full_spec((H, H)),
                  _full_spec((H, H)), _full_spec((1, H))],
        out_specs=[_row_spec(BN, H), _row_spec(BN, H)],
        out_shape=[jax.ShapeDtypeStruct((N, H), jnp.float32),
                   jax.ShapeDtypeStruct((N, H), jnp.float32)],
    )(acc, bcr, W1t, W1b, b1r)


def _tc_ae(ea128, Wd):
    R = E // 8
    BR = R // 5
    return pl.pallas_call(
        _tc_ae_body,
        grid=(5,),
        in_specs=[_row_spec(BR, DF), _full_spec((DF, 16))],
        out_specs=_row_spec(BR, 16),
        out_shape=jax.ShapeDtypeStruct((R, 16), jnp.float32),
    )(ea128, Wd)


def kernel(x, edge_index, edge_attr, W0, b0, Wc1, as1, ad1, We1, ae1, bc1,
           Wc2, as2, ad2, We2, ae2, bc2, W1, b1, W2, b2):
    src = edge_index[0]
    dst = edge_index[1]

    b0r = b0.reshape(1, H)
    bc1r = bc1.reshape(1, H)
    bc2r = bc2.reshape(1, H)
    b1r = b1.reshape(1, H)
    as1r = as1.reshape(1, H)
    ad1r = ad1.reshape(1, H)
    as2r = as2.reshape(1, H)
    ad2r = ad2.reshape(1, H)
    w1e = We1 @ ae1
    w2e = We2 @ ae2
    rows = jnp.arange(DF)
    arow = rows // DE
    krow = rows % DE
    Wd = jnp.zeros((DF, 16), jnp.float32)
    Wd = Wd.at[rows, arow].set(w1e[krow])
    Wd = Wd.at[rows, 8 + arow].set(w2e[krow])
    ea128 = edge_attr.reshape(E // 8, DF)
    W1t = W1[:H]
    W1b = W1[H:]
    wp = jnp.concatenate(
        [W2.reshape(H), jnp.zeros((16,), jnp.float32)])
    b2r = b2.reshape(1, 1)

    ht1, asrc1, adst1 = _tc_encode(x, W0, b0r, Wc1, as1r, ad1r)
    aeo = _tc_ae(ea128, Wd)
    ae1v = aeo[:, :8].reshape(E)
    ae2v = aeo[:, 8:].reshape(E)

    npad = E2 - E
    srcp = jnp.concatenate([src, jnp.zeros((npad,), jnp.int32)])
    dstg = jnp.concatenate([dst, jnp.full((npad,), NP - 1, jnp.int32)])
    dstd = jnp.concatenate([dst, jnp.zeros((npad,), jnp.int32)])
    zpad = jnp.zeros((npad,), jnp.float32)
    ae1p = jnp.concatenate([ae1v, zpad])
    ae2p = jnp.concatenate([ae2v, zpad])
    zro = jnp.zeros((RPT, HP), jnp.float32)

    acc1 = _gat_edge_pass(srcp, dstg, ae1p, asrc1.reshape(N),
                          adst1.reshape(N), ht1, zro)
    ht2, asrc2, adst2 = _tc_mid(acc1, bc1r, Wc2, as2r, ad2r)
    acc2 = _gat_edge_pass(srcp, dstg, ae2p, asrc2.reshape(N),
                          adst2.reshape(N), ht2, zro)
    u, v = _tc_final(acc2, bc2r, W1t, W1b, b1r)
    ug = u.reshape(N, NS, 4).transpose(1, 0, 2).reshape(-1)
    vg = v.reshape(N, NS, 4).transpose(1, 0, 2).reshape(-1)
    partials = _decode_pass(srcp, dstd, ug, vg, wp)
    return _tc_gsum(partials, b2r).reshape(E2)[:E]


# speedup vs baseline: 1.5326x; 1.0305x over previous
import functools

import jax
import jax.numpy as jnp
from jax import lax
from jax.experimental import pallas as pl
from jax.experimental.pallas import tpu as pltpu
from jax.experimental.pallas import tpu_sc as plsc

N = 10000
E = 320000
DF = 128
DE = 16
H = 64
HP = 80
NC = 2
NS = 16
NW = NC * NS
CE = 128
CPT = 80
E2 = NW * CPT * CE
NP = 10240
RPT = NP // NS

_sc_mesh = plsc.VectorSubcoreMesh(core_axis_name="c", subcore_axis_name="s")
_sc_params = pltpu.CompilerParams(needs_layout_passes=False,
                                  use_tc_tiling_on_sc=False)


NSLOT = 4

_GAT_SCRATCH = []
for _ in range(NSLOT):
    _GAT_SCRATCH += [
        pltpu.VMEM((CE,), jnp.int32),
        pltpu.VMEM((CE,), jnp.int32),
        pltpu.VMEM((CE,), jnp.float32),
        pltpu.VMEM((CE,), jnp.float32),
        pltpu.VMEM((CE, HP), jnp.float32),
        pltpu.SemaphoreType.DMA,
        pltpu.SemaphoreType.DMA,
        pltpu.SemaphoreType.DMA,
    ]


@functools.partial(
    pl.kernel,
    out_type=jax.ShapeDtypeStruct((NC, NP, HP), jnp.float32),
    mesh=_sc_mesh,
    compiler_params=_sc_params,
    scratch_types=[
        pltpu.VMEM((N,), jnp.float32),
        pltpu.VMEM((N,), jnp.float32),
        pltpu.VMEM_SHARED((NP, HP), jnp.float32),
    ] + _GAT_SCRATCH,
)
def _gat_edge_pass(src_hbm, dst_hbm, ae_hbm, asrc_hbm, adst_hbm, ht_hbm,
                   zro_hbm, out_hbm, asrc_v, adst_v, acc_sh, *slots):
    c = lax.axis_index("c")
    s = lax.axis_index("s")
    w = s * NC + c
    S = [slots[8 * i:8 * i + 8] for i in range(NSLOT)]

    pltpu.sync_copy(zro_hbm, acc_sh.at[pl.ds(s * RPT, RPT)])
    pltpu.sync_copy(asrc_hbm, asrc_v)
    pltpu.sync_copy(adst_hbm, adst_v)
    plsc.subcore_barrier()

    def _base(j):
        return (w + j * NW) * CE

    def _load(j, b):
        sv, dv, av = S[b][0], S[b][1], S[b][2]
        off = _base(j)
        sem = S[b][7]
        pltpu.async_copy(src_hbm.at[pl.ds(off, CE)], sv, sem)
        pltpu.async_copy(dst_hbm.at[pl.ds(off, CE)], dv, sem)
        pltpu.async_copy(ae_hbm.at[pl.ds(off, CE)], av, sem)

    def _wait_load(j, b):
        sv, dv, av = S[b][0], S[b][1], S[b][2]
        off = _base(j)
        sem = S[b][7]
        pltpu.make_async_copy(src_hbm.at[pl.ds(off, CE)], sv, sem).wait()
        pltpu.make_async_copy(dst_hbm.at[pl.ds(off, CE)], dv, sem).wait()
        pltpu.make_async_copy(ae_hbm.at[pl.ds(off, CE)], av, sem).wait()

    def _gather(b):
        pltpu.async_copy(ht_hbm.at[S[b][0]], S[b][4], S[b][5])

    def _wait_gather(b):
        pltpu.make_async_copy(ht_hbm.at[S[b][0]], S[b][4], S[b][5]).wait()

    def _scatter(b):
        pltpu.async_copy(S[b][4], acc_sh.at[S[b][1]], S[b][6], add=True)

    def _wait_scatter(b):
        pltpu.make_async_copy(S[b][4], acc_sh.at[S[b][1]], S[b][6]).wait()

    def _ex_compute(b):
        sv, dv, av, xv = S[b][0], S[b][1], S[b][2], S[b][3]
        for g in range(CE // 16):
            sl = pl.ds(g * 16, 16)
            lg = (plsc.load_gather(asrc_v, [sv[sl]])
                  + plsc.load_gather(adst_v, [dv[sl]]) + av[sl])
            lg = jnp.where(lg > 0, lg, 0.2 * lg)
            xv[sl] = jnp.exp(lg)

    def _scale(b):
        rv, xv = S[b][4], S[b][3]

        def body(e, carry):
            m = plsc.load_gather(xv, [jnp.full((16,), e, jnp.int32)])
            for q in range(5):
                sl2 = pl.ds(q * 16, 16)
                rv[e, sl2] = rv[e, sl2] * m
            return carry
        lax.fori_loop(0, CE, body, 0)

    for j in range(2):
        _load(j, j)
        _wait_load(j, j)
        _gather(j)

    def _step(j, b):
        _wait_gather(b)
        _ex_compute(b)
        _scale(b)
        _scatter(b)
        b2 = (b + 2) % NSLOT

        @pl.when(j < CPT - 2)
        def _():
            @pl.when(j >= 2)
            def _():
                _wait_scatter(b2)
            _load(j + 2, b2)
            _wait_load(j + 2, b2)
            _gather(b2)

    def _quad_body(p, carry):
        for b in range(NSLOT):
            _step(4 * p + b, b)
        return carry

    lax.fori_loop(0, CPT // NSLOT, _quad_body, 0)
    for b in range(NSLOT):
        _wait_scatter(b)
    plsc.subcore_barrier()

    r0 = s * RPT
    pltpu.sync_copy(acc_sh.at[pl.ds(r0, RPT)], out_hbm.at[c, pl.ds(r0, RPT)])


CED = 2048
EHALF = E2 // 2


@functools.partial(
    pl.kernel,
    out_type=jax.ShapeDtypeStruct((NC, NS, EHALF), jnp.float32),
    mesh=_sc_mesh,
    compiler_params=_sc_params,
    scratch_types=[
        pltpu.VMEM((4 * N,), jnp.float32),
        pltpu.VMEM((4 * N,), jnp.float32),
        pltpu.VMEM((CED,), jnp.int32),
        pltpu.VMEM((CED,), jnp.int32),
        pltpu.VMEM((CED,), jnp.float32),
        pltpu.VMEM((CED,), jnp.int32),
        pltpu.VMEM((CED,), jnp.int32),
        pltpu.VMEM((CED,), jnp.float32),
        pltpu.VMEM((80,), jnp.float32),
        pltpu.SemaphoreType.DMA,
        pltpu.SemaphoreType.DMA,
        pltpu.SemaphoreType.DMA,
        pltpu.SemaphoreType.DMA,
    ],
)
def _decode_pass(src_hbm, dst_hbm, ug_hbm, vg_hbm, wp_hbm, out_hbm,
                 u_t, v_t, src_a, dst_a, out_a, src_b, dst_b, out_b,
                 wp_v, sla, slb, ssa, ssb):
    c = lax.axis_index("c")
    s = lax.axis_index("s")
    pltpu.sync_copy(wp_hbm, wp_v)
    pltpu.sync_copy(ug_hbm.at[pl.ds(s * 4 * N, 4 * N)], u_t)
    pltpu.sync_copy(vg_hbm.at[pl.ds(s * 4 * N, 4 * N)], v_t)
    wks = [plsc.load_gather(wp_v, [jnp.full((16,), 0, jnp.int32) + s * 4 + k])
           for k in range(4)]

    def _eoff(j):
        return c * EHALF + j * CED

    def _load(j, sv, dv, sem):
        b = _eoff(j)
        pltpu.async_copy(src_hbm.at[pl.ds(b, CED)], sv, sem)
        pltpu.async_copy(dst_hbm.at[pl.ds(b, CED)], dv, sem)

    def _wait_load(j, sv, dv, sem):
        b = _eoff(j)
        pltpu.make_async_copy(src_hbm.at[pl.ds(b, CED)], sv, sem).wait()
        pltpu.make_async_copy(dst_hbm.at[pl.ds(b, CED)], dv, sem).wait()

    def _compute(sv, dv, ov):
        def grp(i, carry):
            sl = pl.ds(i * 16, 16)
            si = sv[sl]
            di = dv[sl]
            acc = jnp.zeros((16,), jnp.float32)
            si4 = si * 4
            di4 = di * 4
            for k in range(4):
                uk = plsc.load_gather(u_t, [si4 + k])
                vk = plsc.load_gather(v_t, [di4 + k])
                acc = acc + jnp.maximum(uk + vk, 0.0) * wks[k]
            ov[sl] = acc
            return carry
        lax.fori_loop(0, CED // 16, grp, 0, unroll=2)

    def _store(j, ov, sem):
        pltpu.async_copy(ov, out_hbm.at[c, s, pl.ds(j * CED, CED)], sem)

    def _wait_store(j, ov, sem):
        pltpu.make_async_copy(ov, out_hbm.at[c, s, pl.ds(j * CED, CED)],
                              sem).wait()

    nch = EHALF // CED

    _load(0, src_a, dst_a, sla)
    _wait_load(0, src_a, dst_a, sla)

    def _pair_body(p, carry):
        _load(2 * p + 1, src_b, dst_b, slb)

        @pl.when(p > 0)
        def _():
            _wait_store(2 * p - 2, out_a, ssa)
        _compute(src_a, dst_a, out_a)
        _store(2 * p, out_a, ssa)
        _wait_load(2 * p + 1, src_b, dst_b, slb)

        @pl.when(p < nch // 2 - 1)
        def _():
            _load(2 * p + 2, src_a, dst_a, sla)

        @pl.when(p > 0)
        def _():
            _wait_store(2 * p - 1, out_b, ssb)
        _compute(src_b, dst_b, out_b)
        _store(2 * p + 1, out_b, ssb)

        @pl.when(p < nch // 2 - 1)
        def _():
            _wait_load(2 * p + 2, src_a, dst_a, sla)
        return carry

    lax.fori_loop(0, nch // 2, _pair_body, 0)
    _wait_store(nch - 2, out_a, ssa)
    _wait_store(nch - 1, out_b, ssb)


BN = 2000


def _tc_encode_body(x_ref, W0_ref, b0_ref, Wc_ref, as_ref, ad_ref,
                    ht_ref, asrc_ref, adst_ref):
    z = jnp.dot(x_ref[...], W0_ref[...],
                preferred_element_type=jnp.float32) + b0_ref[...]
    h = jnp.dot(z, Wc_ref[...], preferred_element_type=jnp.float32)
    pad = jnp.concatenate(
        [h, jnp.ones((h.shape[0], 1), jnp.float32),
         jnp.zeros((h.shape[0], HP - H - 1), jnp.float32)], axis=1)
    ht_ref[...] = pad
    asrc_ref[...] = jnp.sum(h * as_ref[...], axis=1, keepdims=True)
    adst_ref[...] = jnp.sum(h * ad_ref[...], axis=1, keepdims=True)


def _tc_mid_body(acc_ref, bc_ref, Wc_ref, as_ref, ad_ref,
                 ht_ref, asrc_ref, adst_ref):
    a = acc_ref[0] + acc_ref[1]
    den = a[:, H:H + 1]
    z = jnp.maximum(a[:, :H] / (den + 1e-16) + bc_ref[...], 0.0)
    h = jnp.dot(z, Wc_ref[...], preferred_element_type=jnp.float32)
    pad = jnp.concatenate(
        [h, jnp.ones((h.shape[0], 1), jnp.float32),
         jnp.zeros((h.shape[0], HP - H - 1), jnp.float32)], axis=1)
    ht_ref[...] = pad
    asrc_ref[...] = jnp.sum(h * as_ref[...], axis=1, keepdims=True)
    adst_ref[...] = jnp.sum(h * ad_ref[...], axis=1, keepdims=True)


def _tc_final_body(acc_ref, bc_ref, W1t_ref, W1b_ref, b1_ref, u_ref, v_ref):
    a = acc_ref[0] + acc_ref[1]
    den = a[:, H:H + 1]
    z = jnp.maximum(a[:, :H] / (den + 1e-16) + bc_ref[...], 0.0)
    u_ref[...] = jnp.dot(z, W1t_ref[...],
                         preferred_element_type=jnp.float32) + b1_ref[...]
    v_ref[...] = jnp.dot(z, W1b_ref[...], preferred_element_type=jnp.float32)


def _tc_gsum_body(p_ref, b2_ref, out_ref):
    out_ref[...] = jax.nn.sigmoid(jnp.concatenate(
        [jnp.sum(p_ref[0], axis=0, keepdims=True),
         jnp.sum(p_ref[1], axis=0, keepdims=True)], axis=0) + b2_ref[...])


def _tc_gsum(partials, b2r):
    BE2 = EHALF // 10
    return pl.pallas_call(
        _tc_gsum_body,
        grid=(10,),
        in_specs=[pl.BlockSpec((NC, NS, BE2), lambda i: (0, 0, i)),
                  _full_spec((1, 1))],
        out_specs=pl.BlockSpec((NC, BE2), lambda i: (0, i)),
        out_shape=jax.ShapeDtypeStruct((NC, EHALF), jnp.float32),
    )(partials, b2r)


def _tc_ae_body(ea_ref, Wd_ref, out_ref):
    out_ref[...] = jnp.dot(ea_ref[...], Wd_ref[...],
                           preferred_element_type=jnp.float32)


def _row_spec(bn, ncols):
    return pl.BlockSpec((bn, ncols), lambda i: (i, 0))


def _full_spec(shape):
    return pl.BlockSpec(shape, lambda i: tuple(0 for _ in shape))


def _tc_encode(x, W0, b0r, Wc, asr, adr):
    grid = (N // BN,)
    return pl.pallas_call(
        _tc_encode_body,
        grid=grid,
        in_specs=[_row_spec(BN, DF), _full_spec((DF, H)), _full_spec((1, H)),
                  _full_spec((H, H)), _full_spec((1, H)), _full_spec((1, H))],
        out_specs=[_row_spec(BN, HP),
                   _row_spec(BN, 1), _row_spec(BN, 1)],
        out_shape=[jax.ShapeDtypeStruct((N, HP), jnp.float32),
                   jax.ShapeDtypeStruct((N, 1), jnp.float32),
                   jax.ShapeDtypeStruct((N, 1), jnp.float32)],
    )(x, W0, b0r, Wc, asr, adr)


def _tc_mid(acc, bcr, Wc, asr, adr):
    grid = (N // BN,)
    return pl.pallas_call(
        _tc_mid_body,
        grid=grid,
        in_specs=[pl.BlockSpec((NC, BN, HP), lambda i: (0, i, 0)),
                  _full_spec((1, H)), _full_spec((H, H)),
                  _full_spec((1, H)), _full_spec((1, H))],
        out_specs=[_row_spec(BN, HP),
                   _row_spec(BN, 1), _row_spec(BN, 1)],
        out_shape=[jax.ShapeDtypeStruct((N, HP), jnp.float32),
                   jax.ShapeDtypeStruct((N, 1), jnp.float32),
                   jax.ShapeDtypeStruct((N, 1), jnp.float32)],
    )(acc, bcr, Wc, asr, adr)


def _tc_final(acc, bcr, W1t, W1b, b1r):
    grid = (N // BN,)
    return pl.pallas_call(
        _tc_final_body,
        grid=grid,
        in_specs=[pl.BlockSpec((NC, BN, HP), lambda i: (0, i, 0)),
                  _full_spec((1, H)), _full_spec((H, H)),
                  _full_spec((H, H)), _full_spec((1, H))],
        out_specs=[_row_spec(BN, H), _row_spec(BN, H)],
        out_shape=[jax.ShapeDtypeStruct((N, H), jnp.float32),
                   jax.ShapeDtypeStruct((N, H), jnp.float32)],
    )(acc, bcr, W1t, W1b, b1r)


def _tc_ae(ea128, Wd):
    R = E // 8
    BR = R // 5
    return pl.pallas_call(
        _tc_ae_body,
        grid=(5,),
        in_specs=[_row_spec(BR, DF), _full_spec((DF, 16))],
        out_specs=_row_spec(BR, 16),
        out_shape=jax.ShapeDtypeStruct((R, 16), jnp.float32),
    )(ea128, Wd)


def kernel(x, edge_index, edge_attr, W0, b0, Wc1, as1, ad1, We1, ae1, bc1,
           Wc2, as2, ad2, We2, ae2, bc2, W1, b1, W2, b2):
    src = edge_index[0]
    dst = edge_index[1]

    b0r = b0.reshape(1, H)
    bc1r = bc1.reshape(1, H)
    bc2r = bc2.reshape(1, H)
    b1r = b1.reshape(1, H)
    as1r = as1.reshape(1, H)
    ad1r = ad1.reshape(1, H)
    as2r = as2.reshape(1, H)
    ad2r = ad2.reshape(1, H)
    w1e = We1 @ ae1
    w2e = We2 @ ae2
    rows = jnp.arange(DF)
    arow = rows // DE
    krow = rows % DE
    Wd = jnp.zeros((DF, 16), jnp.float32)
    Wd = Wd.at[rows, arow].set(w1e[krow])
    Wd = Wd.at[rows, 8 + arow].set(w2e[krow])
    ea128 = edge_attr.reshape(E // 8, DF)
    W1t = W1[:H]
    W1b = W1[H:]
    wp = jnp.concatenate(
        [W2.reshape(H), jnp.zeros((16,), jnp.float32)])
    b2r = b2.reshape(1, 1)

    ht1, asrc1, adst1 = _tc_encode(x, W0, b0r, Wc1, as1r, ad1r)
    aeo = _tc_ae(ea128, Wd)
    ae1v = aeo[:, :8].reshape(E)
    ae2v = aeo[:, 8:].reshape(E)

    npad = E2 - E
    srcp = jnp.concatenate([src, jnp.zeros((npad,), jnp.int32)])
    dstg = jnp.concatenate([dst, jnp.full((npad,), NP - 1, jnp.int32)])
    dstd = jnp.concatenate([dst, jnp.zeros((npad,), jnp.int32)])
    zpad = jnp.zeros((npad,), jnp.float32)
    ae1p = jnp.concatenate([ae1v, zpad])
    ae2p = jnp.concatenate([ae2v, zpad])
    zro = jnp.zeros((RPT, HP), jnp.float32)

    acc1 = _gat_edge_pass(srcp, dstg, ae1p, asrc1.reshape(N),
                          adst1.reshape(N), ht1, zro)
    ht2, asrc2, adst2 = _tc_mid(acc1, bc1r, Wc2, as2r, ad2r)
    acc2 = _gat_edge_pass(srcp, dstg, ae2p, asrc2.reshape(N),
                          adst2.reshape(N), ht2, zro)
    u, v = _tc_final(acc2, bc2r, W1t, W1b, b1r)
    ug = u.reshape(N, NS, 4).transpose(1, 0, 2).reshape(-1)
    vg = v.reshape(N, NS, 4).transpose(1, 0, 2).reshape(-1)
    partials = _decode_pass(srcp, dstd, ug, vg, wp)
    return _tc_gsum(partials, b2r).reshape(E2)[:E]
